# baseline (speedup 1.0000x reference)
import jax
import jax.numpy as jnp
from jax.experimental import pallas as pl
from jax.experimental.pallas import tpu as pltpu

N_USERS = 4096
N_POSTS = 4096
N_ALL = N_USERS + N_POSTS
BATCH = 4096
H = 128

_RM = 256
_NB = N_ALL // _RM
_NBU = N_USERS // _RM

_INTERPRET = False


def _gnn_kernel(uf_ref, pf_ref, adj_ref, ui_ref, pi_ref,
                wu_ref, bu_ref, wp_ref, bp_ref,
                wg0_ref, bg0_ref, wg1_ref, bg1_ref,
                w0u_ref, w0p_ref, b0_ref, w1_ref, b1_ref, w2_ref, b2_ref,
                out_ref, s0_ref, s1_ref, cu_ref, cp_ref, du_ref, dp_ref):
    f32 = jnp.float32
    i = pl.program_id(0)

    @pl.when(i == 0)
    def _proj():
        for h, (f_ref, w_ref, b_ref) in enumerate(
                ((uf_ref, wu_ref, bu_ref), (pf_ref, wp_ref, bp_ref))):
            emb = jnp.dot(f_ref[...], w_ref[...],
                          preferred_element_type=f32) + b_ref[...]
            s0_ref[pl.ds(h * N_USERS, N_USERS), :] = jnp.dot(
                emb, wg0_ref[...], preferred_element_type=f32)
        cu_ref[...] = jnp.zeros(cu_ref.shape, f32)
        cp_ref[...] = jnp.zeros(cp_ref.shape, f32)
        iota = jax.lax.broadcasted_iota(jnp.int32, (BATCH, _RM), 1)
        du_ref[...] = jnp.broadcast_to(ui_ref[...], (BATCH, _RM)) - iota
        dp_ref[...] = jnp.broadcast_to(pi_ref[...], (BATCH, _RM)) - iota

    @pl.when((i >= 1) & (i <= _NB))
    def _l1():
        acc = jnp.dot(adj_ref[...], s0_ref[...], preferred_element_type=f32)
        h1 = jnp.maximum(acc + bg0_ref[...], 0.0)
        s1_ref[pl.ds((i - 1) * _RM, _RM), :] = jnp.dot(
            h1, wg1_ref[...], preferred_element_type=f32)

    @pl.when((i > _NB) & (i <= 2 * _NB))
    def _l2():
        b_loc = i - _NB - 1
        acc = jnp.dot(adj_ref[...], s1_ref[...], preferred_element_type=f32)
        h2_blk = jnp.maximum(acc + bg1_ref[...], 0.0)

        @pl.when(b_loc < _NBU)
        def _users():
            oh = (du_ref[...] == b_loc * _RM).astype(f32)
            cu_ref[...] += jnp.dot(oh, h2_blk, preferred_element_type=f32)

        @pl.when(b_loc >= _NBU)
        def _posts():
            oh = (dp_ref[...] == b_loc * _RM - N_USERS).astype(f32)
            cp_ref[...] += jnp.dot(oh, h2_blk, preferred_element_type=f32)

    @pl.when(i == 2 * _NB + 1)
    def _head():
        x = (jnp.dot(cu_ref[...], w0u_ref[...], preferred_element_type=f32)
             + jnp.dot(cp_ref[...], w0p_ref[...], preferred_element_type=f32)
             + b0_ref[...])
        x = jnp.maximum(x, 0.0)
        x = jnp.maximum(
            jnp.dot(x, w1_ref[...], preferred_element_type=f32) + b1_ref[...],
            0.0)
        s = jnp.dot(x, w2_ref[...], preferred_element_type=f32) + b2_ref[...]
        out_ref[...] = jax.nn.sigmoid(s)


def kernel(user_features, post_features, adj_matrix, user_indices, post_indices,
           Wu, bu, Wp, bp, Wg0, bg0, Wg1, bg1, Wh0, bh0, Wh1, bh1, Wh2, bh2):
    f32 = jnp.float32
    d_in = user_features.shape[1]
    adj_map = lambda i: (
        jnp.where(i > _NB, jnp.minimum(i - _NB - 1, _NB - 1),
                  jnp.maximum(i - 1, 0)), 0)
    const2 = lambda i: (0, 0)
    scores = pl.pallas_call(
        _gnn_kernel,
        grid=(2 * _NB + 2,),
        in_specs=[
            pl.BlockSpec((N_USERS, d_in), const2),
            pl.BlockSpec((N_POSTS, d_in), const2),
            pl.BlockSpec((_RM, N_ALL), adj_map),
            pl.BlockSpec((BATCH, 1), const2),
            pl.BlockSpec((BATCH, 1), const2),
            pl.BlockSpec((d_in, H), const2),
            pl.BlockSpec((1, H), const2),
            pl.BlockSpec((d_in, H), const2),
            pl.BlockSpec((1, H), const2),
            pl.BlockSpec((H, H), const2),
            pl.BlockSpec((1, H), const2),
            pl.BlockSpec((H, H), const2),
            pl.BlockSpec((1, H), const2),
            pl.BlockSpec((H, H), const2),
            pl.BlockSpec((H, H), const2),
            pl.BlockSpec((1, H), const2),
            pl.BlockSpec((H, H // 2), const2),
            pl.BlockSpec((1, H // 2), const2),
            pl.BlockSpec((H // 2, 1), const2),
            pl.BlockSpec((1, 1), const2),
        ],
        out_specs=pl.BlockSpec((BATCH, 1), const2),
        out_shape=jax.ShapeDtypeStruct((BATCH, 1), f32),
        scratch_shapes=[
            pltpu.VMEM((N_ALL, H), f32),
            pltpu.VMEM((N_ALL, H), f32),
            pltpu.VMEM((BATCH, H), f32),
            pltpu.VMEM((BATCH, H), f32),
            pltpu.VMEM((BATCH, _RM), jnp.int32),
            pltpu.VMEM((BATCH, _RM), jnp.int32),
        ],
        interpret=_INTERPRET,
    )(user_features, post_features, adj_matrix,
      user_indices.astype(jnp.int32).reshape(BATCH, 1),
      post_indices.astype(jnp.int32).reshape(BATCH, 1),
      Wu, bu.reshape(1, H), Wp, bp.reshape(1, H),
      Wg0, bg0.reshape(1, H), Wg1, bg1.reshape(1, H),
      Wh0[:H], Wh0[H:], bh0.reshape(1, H),
      Wh1, bh1.reshape(1, H // 2), Wh2, bh2.reshape(1, 1))
    return jnp.squeeze(scores, axis=-1)
